# baseline (speedup 1.0000x reference)
import jax
import jax.numpy as jnp
from jax.experimental import pallas as pl
from jax.experimental.pallas import tpu as pltpu

_BM = 400


def _gcn_kernel(x_ref, adj_ref, w_ref, out_ref):
    b = x_ref.shape[0]
    adj_tile = adj_ref[0]
    for i in range(b):
        agg = jnp.dot(adj_tile, x_ref[i], preferred_element_type=jnp.float32)
        out_ref[i] = jnp.maximum(
            jnp.dot(agg, w_ref[...], preferred_element_type=jnp.float32), 0.0)


def kernel(inputs, adj, W, W_agg):
    B, N, D = inputs.shape
    del W_agg

    adj3 = adj.reshape(N // _BM, _BM, N)
    out = pl.pallas_call(
        _gcn_kernel,
        grid=(N // _BM,),
        in_specs=[
            pl.BlockSpec((B, N, D), lambda i: (0, 0, 0)),
            pl.BlockSpec((1, _BM, N), lambda i: (i, 0, 0)),
            pl.BlockSpec((D, D), lambda i: (0, 0)),
        ],
        out_specs=pl.BlockSpec((B, _BM, D), lambda i: (0, i, 0)),
        out_shape=jax.ShapeDtypeStruct((B, N, D), jnp.float32),
        compiler_params=pltpu.CompilerParams(
            dimension_semantics=("parallel",)),
    )(inputs, adj3, W)

    return out


# speedup vs baseline: 1.0093x; 1.0093x over previous
import jax
import jax.numpy as jnp
from jax.experimental import pallas as pl
from jax.experimental.pallas import tpu as pltpu

_BM = 400


def _gcn_kernel(x_ref, adj_ref, w_ref, out_ref):
    b = x_ref.shape[0]
    adj_tile = adj_ref[0]
    for i in range(b):
        agg = jnp.dot(adj_tile, x_ref[i], preferred_element_type=jnp.float32)
        out_ref[i] = jnp.maximum(
            jnp.dot(agg, w_ref[...], preferred_element_type=jnp.float32), 0.0)


def kernel(inputs, adj, W, W_agg):
    B, N, D = inputs.shape
    del W_agg

    adj3 = adj.reshape(N // _BM, _BM, N)
    out = pl.pallas_call(
        _gcn_kernel,
        grid=(N // _BM,),
        in_specs=[
            pl.BlockSpec((B, N, D), lambda i: (0, 0, 0)),
            pl.BlockSpec((1, _BM, N), lambda i: (i, 0, 0)),
            pl.BlockSpec((D, D), lambda i: (0, 0)),
        ],
        out_specs=pl.BlockSpec((B, _BM, D), lambda i: (0, i, 0)),
        out_shape=jax.ShapeDtypeStruct((B, N, D), jnp.float32),
        compiler_params=pltpu.CompilerParams(
            dimension_semantics=("arbitrary",)),
    )(inputs, adj3, W)

    return out


# speedup vs baseline: 1.7179x; 1.7020x over previous
import jax
import jax.numpy as jnp
from jax.experimental import pallas as pl
from jax.experimental.pallas import tpu as pltpu

_BM = 400


def _fused_kernel(x_ref, adj_ref, w_ref, out_ref, ps_ref):
    b, _, d = x_ref.shape

    @pl.when(pl.program_id(0) == 0)
    def _compute_presup():
        for i in range(b):
            ps_ref[:, i * d:(i + 1) * d] = jnp.dot(
                x_ref[i], w_ref[...],
                preferred_element_type=jnp.float32).astype(jnp.bfloat16)

    acc = jnp.dot(adj_ref[0].astype(jnp.bfloat16), ps_ref[...],
                  preferred_element_type=jnp.float32)
    acc = jnp.maximum(acc, 0.0)
    for i in range(b):
        out_ref[i] = acc[:, i * d:(i + 1) * d]


def kernel(inputs, adj, W, W_agg):
    B, N, D = inputs.shape
    del W_agg

    adj3 = adj.reshape(N // _BM, _BM, N)
    out = pl.pallas_call(
        _fused_kernel,
        grid=(N // _BM,),
        in_specs=[
            pl.BlockSpec((B, N, D), lambda i: (0, 0, 0)),
            pl.BlockSpec((1, _BM, N), lambda i: (i, 0, 0)),
            pl.BlockSpec((D, D), lambda i: (0, 0)),
        ],
        out_specs=pl.BlockSpec((B, _BM, D), lambda i: (0, i, 0)),
        out_shape=jax.ShapeDtypeStruct((B, N, D), jnp.float32),
        scratch_shapes=[pltpu.VMEM((N, B * D), jnp.bfloat16)],
        compiler_params=pltpu.CompilerParams(
            dimension_semantics=("arbitrary",)),
    )(inputs, adj3, W)

    return out
